# Initial kernel scaffold; baseline (speedup 1.0000x reference)
import jax
import jax.numpy as jnp
from jax.experimental import pallas as pl


def kernel(x):
    raise NotImplementedError("write your pallas kernel here")


# speedup vs baseline: 1.5124x; 1.5124x over previous
import functools

import jax
import jax.numpy as jnp
from jax import lax
from jax.experimental import pallas as pl
from jax.experimental.pallas import tpu as pltpu
from jax.experimental.pallas import tpu_sc as plsc

_B, _T = 16, 160000
_N = _B * _T
_NC, _NS = 2, 16
_NW = _NC * _NS
_PER = _N // _NW

_mesh = plsc.VectorSubcoreMesh(core_axis_name="c", subcore_axis_name="s")


@functools.partial(
    pl.kernel,
    out_type=jax.ShapeDtypeStruct((_N,), jnp.float32),
    mesh=_mesh,
    scratch_types=[pltpu.VMEM((_PER,), jnp.float32)],
)
def _sc_copy(x_hbm, out_hbm, buf):
    wid = lax.axis_index("s") * _NC + lax.axis_index("c")
    base = wid * _PER
    pltpu.sync_copy(x_hbm.at[pl.ds(base, _PER)], buf)
    pltpu.sync_copy(buf, out_hbm.at[pl.ds(base, _PER)])


def kernel(x):
    return _sc_copy(x.reshape(_N)).reshape(_B, _T)
